# Initial kernel scaffold; baseline (speedup 1.0000x reference)
import jax
import jax.numpy as jnp
from jax.experimental import pallas as pl


def kernel(frame_emb, beat_bounds, W, b):
    raise NotImplementedError("write your pallas kernel here")


# speedup vs baseline: 1.5891x; 1.5891x over previous
import math

import jax
import jax.numpy as jnp
from jax.experimental import pallas as pl
from jax.experimental.pallas import tpu as pltpu

D_MODEL_ = 256
POS_DIM_ = 32


def _fourier_table(M, dtype):
    half = POS_DIM_ // 2
    freqs = jnp.exp(jnp.linspace(math.log(1.0), math.log(1000.0), half))
    idx = jnp.arange(M, dtype=dtype)
    pos = jnp.clip(idx / max(1, M - 1), 0.0, 1.0)
    ang = pos[:, None] * freqs
    out = jnp.concatenate([jnp.sin(ang), jnp.cos(ang)], axis=-1)
    if out.shape[-1] < POS_DIM_:
        out = jnp.concatenate(
            [out, jnp.zeros(out.shape[:-1] + (POS_DIM_ - out.shape[-1],), out.dtype)],
            axis=-1)
    return out.astype(dtype)


def _pool_kernel(bounds_ref, x_ref, w_ref, bias_ref, ff_ref, o_ref):
    T = x_ref.shape[1]
    M = bounds_ref.shape[1]
    s = bounds_ref[0, :, 0]
    e = bounds_ref[0, :, 1]
    s = jnp.clip(s, 0, T - 1)
    e = jnp.minimum(e, T)
    e = jnp.maximum(s + 1, e)

    t = jax.lax.broadcasted_iota(jnp.int32, (M, T), 1)
    mask = (t >= s[:, None]) & (t < e[:, None])
    maskf = mask.astype(jnp.float32)

    sums = jnp.dot(maskf, x_ref[0], preferred_element_type=jnp.float32)
    inv = 1.0 / (e - s).astype(jnp.float32)
    mean = sums * inv[:, None]

    w_top = w_ref[:D_MODEL_, :]
    w_bot = w_ref[D_MODEL_:, :]
    out = jnp.dot(mean, w_top, preferred_element_type=jnp.float32)
    out += jnp.dot(ff_ref[...], w_bot, preferred_element_type=jnp.float32)
    out += bias_ref[...][None, :]
    o_ref[0] = out


def kernel(frame_emb, beat_bounds, W, b):
    B, T, D = frame_emb.shape
    M = beat_bounds.shape[1]
    bounds = beat_bounds.astype(jnp.int32)
    ff = _fourier_table(M, frame_emb.dtype)

    return pl.pallas_call(
        _pool_kernel,
        grid=(B,),
        in_specs=[
            pl.BlockSpec((1, M, 2), lambda i: (i, 0, 0)),
            pl.BlockSpec((1, T, D), lambda i: (i, 0, 0)),
            pl.BlockSpec((D + POS_DIM_, D), lambda i: (0, 0)),
            pl.BlockSpec((D,), lambda i: (0,)),
            pl.BlockSpec((M, POS_DIM_), lambda i: (0, 0)),
        ],
        out_specs=pl.BlockSpec((1, M, D), lambda i: (i, 0, 0)),
        out_shape=jax.ShapeDtypeStruct((B, M, D), frame_emb.dtype),
    )(bounds, frame_emb, W, b, ff)
